# Initial kernel scaffold; baseline (speedup 1.0000x reference)
import jax
import jax.numpy as jnp
from jax.experimental import pallas as pl


def kernel(x, edge_index, edge_attr, history_actions, effect_history_actions, partition, params):
    raise NotImplementedError("write your pallas kernel here")


# speedup vs baseline: 4.4319x; 4.4319x over previous
import functools

import jax
import jax.numpy as jnp
from jax import lax
from jax.experimental import pallas as pl
from jax.experimental.pallas import tpu as pltpu
from jax.experimental.pallas import tpu_sc as plsc

N = 10000
NPAD = 10240
E = 320000
H = 128
PN = 20
ROWW = H
BLK = NPAD // 4
NW = 32
EPW = E // NW
CH = 64
NFULL = EPW // CH
TAIL = EPW - NFULL * CH
RPS = NPAD // 16
DROWS = NPAD // 16

_HI = jax.lax.Precision.HIGHEST


def _dot(a, b):
    return jnp.dot(a, b, precision=_HI)


def _k0_body(x_ref, wl_ref, wr_ref, ea_ref, xl_ref, xr_ref, easum_ref):
    i = pl.program_id(0)
    xb = x_ref[...]
    xl_ref[...] = _dot(xb, wl_ref[...])
    xr_ref[...] = _dot(xb, wr_ref[...])

    @pl.when(i == 0)
    def _():
        easum_ref[...] = jnp.zeros_like(easum_ref)

    easum_ref[...] += jnp.sum(ea_ref[...], axis=0, keepdims=True)


def _k0(xp, wl, wr, ear):
    return pl.pallas_call(
        _k0_body,
        grid=(4,),
        in_specs=[
            pl.BlockSpec((BLK, H), lambda i: (i, 0)),
            pl.BlockSpec((H, H), lambda i: (0, 0)),
            pl.BlockSpec((H, H), lambda i: (0, 0)),
            pl.BlockSpec((BLK, H), lambda i: (i, 0)),
        ],
        out_specs=[
            pl.BlockSpec((BLK, H), lambda i: (i, 0)),
            pl.BlockSpec((BLK, H), lambda i: (i, 0)),
            pl.BlockSpec((1, H), lambda i: (0, 0)),
        ],
        out_shape=[
            jax.ShapeDtypeStruct((NPAD, H), jnp.float32),
            jax.ShapeDtypeStruct((NPAD, H), jnp.float32),
            jax.ShapeDtypeStruct((1, H), jnp.float32),
        ],
    )(xp, wl, wr, ear)


def _layer_out(p_ref, d_ref, xl_ref, xr_ref, easum_ref, we_ref, att_ref,
               b_ref):
    xl = xl_ref[...]
    xr = xr_ref[...]
    sel = (lax.broadcasted_iota(jnp.int32, (H, 4), 0) % 4
           == lax.broadcasted_iota(jnp.int32, (H, 4), 1))
    eam = _dot(easum_ref[...], sel.astype(jnp.float32)) * (1.0 / E)
    eloop = _dot(eam, we_ref[...])
    u = xl + xr + eloop
    m = jnp.maximum(u, 0.2 * u)
    logit = jnp.sum(m * att_ref[...], axis=1, keepdims=True)
    wself = jnp.exp(logit)
    num = p_ref[0] + p_ref[1] + wself * xl
    drow = d_ref[0:1, :] + d_ref[1:2, :]
    dsum = lax.dot_general(drow, jnp.ones((1, 1), jnp.float32),
                           (((0,), (0,)), ((), ())), precision=_HI)
    dn = dsum + wself
    return num / dn + b_ref[...]


def _kmid_body(p_ref, d_ref, xl_ref, xr_ref, easum_ref, we_ref, att_ref,
               b_ref, wl2_ref, wr2_ref, xlo_ref, xro_ref):
    i = pl.program_id(0)
    h = _layer_out(p_ref, d_ref, xl_ref, xr_ref, easum_ref, we_ref, att_ref,
                   b_ref)
    h = jnp.maximum(h, 0.0)
    rowid = lax.broadcasted_iota(jnp.int32, (BLK, H), 0) + i * BLK
    h = jnp.where(rowid < N, h, 0.0)
    xlo_ref[...] = _dot(h, wl2_ref[...])
    xro_ref[...] = _dot(h, wr2_ref[...])


def _kmid(p, d, xl, xr, easum, we, att2, b2, wl2, wr2):
    return pl.pallas_call(
        _kmid_body,
        grid=(4,),
        in_specs=[
            pl.BlockSpec((2, BLK, H), lambda i: (0, i, 0)),
            pl.BlockSpec((2, BLK), lambda i: (0, i)),
            pl.BlockSpec((BLK, H), lambda i: (i, 0)),
            pl.BlockSpec((BLK, H), lambda i: (i, 0)),
            pl.BlockSpec((1, H), lambda i: (0, 0)),
            pl.BlockSpec((4, H), lambda i: (0, 0)),
            pl.BlockSpec((1, H), lambda i: (0, 0)),
            pl.BlockSpec((1, H), lambda i: (0, 0)),
            pl.BlockSpec((H, H), lambda i: (0, 0)),
            pl.BlockSpec((H, H), lambda i: (0, 0)),
        ],
        out_specs=[
            pl.BlockSpec((BLK, H), lambda i: (i, 0)),
            pl.BlockSpec((BLK, H), lambda i: (i, 0)),
        ],
        out_shape=[
            jax.ShapeDtypeStruct((NPAD, H), jnp.float32),
            jax.ShapeDtypeStruct((NPAD, H), jnp.float32),
        ],
    )(p, d, xl, xr, easum, we, att2, b2, wl2, wr2)


def _kfin_body(p_ref, d_ref, xl_ref, xr_ref, easum_ref, we_ref, att_ref,
               b_ref, part_ref, sums_ref, cnts_ref):
    i = pl.program_id(0)
    h = _layer_out(p_ref, d_ref, xl_ref, xr_ref, easum_ref, we_ref, att_ref,
                   b_ref)
    part = part_ref[0]
    pv = lax.broadcasted_iota(jnp.int32, (PN, BLK), 0)
    oh = jnp.where(part == pv, 1.0, 0.0)

    @pl.when(i == 0)
    def _():
        sums_ref[...] = jnp.zeros_like(sums_ref)
        cnts_ref[...] = jnp.zeros_like(cnts_ref)

    sums_ref[...] += _dot(oh, h)
    cnts_ref[...] += jnp.broadcast_to(
        jnp.sum(oh, axis=1, keepdims=True), (PN, H))


def _kfin(p, d, xl, xr, easum, we, att3, b3, partp):
    return pl.pallas_call(
        _kfin_body,
        grid=(4,),
        in_specs=[
            pl.BlockSpec((2, BLK, H), lambda i: (0, i, 0)),
            pl.BlockSpec((2, BLK), lambda i: (0, i)),
            pl.BlockSpec((BLK, H), lambda i: (i, 0)),
            pl.BlockSpec((BLK, H), lambda i: (i, 0)),
            pl.BlockSpec((1, H), lambda i: (0, 0)),
            pl.BlockSpec((4, H), lambda i: (0, 0)),
            pl.BlockSpec((1, H), lambda i: (0, 0)),
            pl.BlockSpec((1, H), lambda i: (0, 0)),
            pl.BlockSpec((1, 1, BLK), lambda i: (i, 0, 0)),
        ],
        out_specs=[
            pl.BlockSpec((PN, H), lambda i: (0, 0)),
            pl.BlockSpec((PN, H), lambda i: (0, 0)),
        ],
        out_shape=[
            jax.ShapeDtypeStruct((PN, H), jnp.float32),
            jax.ShapeDtypeStruct((PN, H), jnp.float32),
        ],
    )(p, d, xl, xr, easum, we, att3, b3, partp)


def _leaky(x):
    return jnp.maximum(x, 0.01 * x)


def _ktail_body(sums_ref, cnts_ref, hist_ref,
                wqt_ref, bq_ref, wkt_ref, bk_ref, wvt_ref, bv_ref,
                wot_ref, bo_ref,
                hw1_ref, hb1_ref, hw2_ref, hb2_ref, hw3_ref, hb3_ref,
                dw1_ref, db1_ref, dw2_ref, db2_ref, dw3_ref, db3_ref,
                out_ref):
    cm = sums_ref[...] / cnts_ref[...]
    q = _dot(cm, wqt_ref[...]) + bq_ref[...]
    k = _dot(cm, wkt_ref[...]) + bk_ref[...]
    v = _dot(cm, wvt_ref[...]) + bv_ref[...]
    hd = H // 4
    heads = []
    for hh in range(4):
        sl = slice(hh * hd, (hh + 1) * hd)
        qh, kh, vh = q[:, sl], k[:, sl], v[:, sl]
        s = lax.dot_general(qh, kh, (((1,), (1,)), ((), ())),
                            precision=_HI) * (1.0 / (hd ** 0.5))
        mx = jnp.max(s, axis=1, keepdims=True)
        ex = jnp.exp(s - mx)
        a = ex / jnp.sum(ex, axis=1, keepdims=True)
        heads.append(_dot(a, vh))
    o128 = jnp.concatenate(heads, axis=1)
    xa = _dot(o128, wot_ref[...]) + bo_ref[...]

    o = _leaky(_dot(hist_ref[...], hw1_ref[...]) + hb1_ref[...])
    o = _leaky(_dot(o, hw2_ref[...]) + hb2_ref[...])
    o = _dot(o, hw3_ref[...]) + hb3_ref[...]

    z = jnp.concatenate([xa, o], axis=0)
    u = db1_ref[...]
    for p_ in range(PN + 1):
        u = u + _dot(z[p_:p_ + 1, :], dw1_ref[p_])
    u = _leaky(u)
    u = _leaky(_dot(u, dw2_ref[...]) + db2_ref[...])
    out_ref[...] = _dot(u, dw3_ref[...]) + db3_ref[...]


def _ktail(sums, cnts, hist2, mm, hp, dp):
    args = [sums, cnts, hist2,
            mm['Wq'].T, mm['bq'].reshape(1, H),
            mm['Wk'].T, mm['bk'].reshape(1, H),
            mm['Wv'].T, mm['bv'].reshape(1, H),
            mm['Wo'].T, mm['bo'].reshape(1, H),
            hp['W1'], hp['b1'].reshape(1, H),
            hp['W2'], hp['b2'].reshape(1, H),
            hp['W3'], hp['b3'].reshape(1, H),
            dp['W1'].reshape(PN + 1, H, H), dp['b1'].reshape(1, H),
            dp['W2'], dp['b2'].reshape(1, H),
            dp['W3'], dp['b3'].reshape(1, PN)]
    return pl.pallas_call(
        _ktail_body,
        in_specs=[pl.BlockSpec(a.shape, functools.partial(lambda nd: (0,) * nd, a.ndim))
                  for a in args],
        out_specs=pl.BlockSpec((1, PN), lambda: (0, 0)),
        out_shape=jax.ShapeDtypeStruct((1, PN), jnp.float32),
    )(*args)


def _sc_body(xl_hbm, xr_hbm, src_hbm, dst_hbm, ea_hbm, we_hbm, att_hbm,
             outn_hbm, outd_hbm,
             xlg, xrg, rows, dnrows, srcv, dstv, dnidx, eav,
             wev, attv, acc, accd, sem):
    c = lax.axis_index("c")
    s = lax.axis_index("s")
    wid = c * 16 + s

    pltpu.sync_copy(we_hbm, wev)
    pltpu.sync_copy(att_hbm, attv)

    zero16 = jnp.zeros((16,), jnp.float32)

    def zrow(r, carry):
        for jj in range(H // 16):
            rows[r, pl.ds(jj * 16, 16)] = zero16
        dnrows[r, :] = zero16
        return carry

    lax.fori_loop(0, CH, zrow, 0)
    base_r = s * RPS
    for kk in range(RPS // CH):
        pltpu.sync_copy(rows, acc.at[pl.ds(base_r + kk * CH, CH)])
    if RPS % CH:
        pltpu.sync_copy(rows.at[pl.ds(0, RPS % CH)],
                        acc.at[pl.ds(base_r + (RPS // CH) * CH, RPS % CH)])

    @pl.when(s == 0)
    def _():
        for kk in range(DROWS // CH):
            pltpu.sync_copy(dnrows, accd.at[pl.ds(kk * CH, CH)])
        if DROWS % CH:
            pltpu.sync_copy(dnrows.at[pl.ds(0, DROWS % CH)],
                            accd.at[pl.ds((DROWS // CH) * CH, DROWS % CH)])

    plsc.subcore_barrier()

    pidx = lax.iota(jnp.int32, 16)

    def process(off, csz):
        b_xlg = xlg.at[pl.ds(0, csz)]
        b_xrg = xrg.at[pl.ds(0, csz)]
        b_rows = rows.at[pl.ds(0, csz)]
        b_dn = dnrows.at[pl.ds(0, csz)]
        b_src = srcv.at[pl.ds(0, csz)]
        b_dst = dstv.at[pl.ds(0, csz)]
        b_di = dnidx.at[pl.ds(0, csz)]
        b_ea = eav.at[pl.ds(0, csz)]
        pltpu.sync_copy(src_hbm.at[pl.ds(off, csz)], b_src)
        pltpu.sync_copy(dst_hbm.at[pl.ds(off, csz)], b_dst)
        pltpu.sync_copy(ea_hbm.at[pl.ds(off, csz)], b_ea)
        pltpu.async_copy(xl_hbm.at[b_src], b_xlg, sem).wait()
        pltpu.async_copy(xr_hbm.at[b_dst], b_xrg, sem).wait()

        def group(g, carry):
            idxv = b_dst[pl.ds(g * 16, 16)]
            b_di[pl.ds(g * 16, 16)] = lax.shift_right_logical(idxv, 4)
            lv = lax.bitwise_and(idxv, 15)
            for j in range(16):
                i = g * 16 + j
                ev = b_ea[i, :]
                a0 = ev[0]
                a1 = ev[1]
                a2 = ev[2]
                a3 = ev[3]
                lacc = zero16
                for jj in range(8):
                    sl = pl.ds(jj * 16, 16)
                    u = b_xlg[i, sl] + b_xrg[i, sl]
                    u = (u + a0 * wev[0, sl] + a1 * wev[1, sl]
                         + a2 * wev[2, sl] + a3 * wev[3, sl])
                    m = jnp.maximum(u, 0.2 * u)
                    lacc = lacc + m * attv[sl]
                for stp in (1, 2, 4, 8):
                    lacc = lacc + lacc.at[pidx ^ stp].get(
                        mode='promise_in_bounds')
                wv = jnp.exp(lacc)
                lj = lv[j]
                for jj in range(8):
                    sl = pl.ds(jj * 16, 16)
                    b_rows[i, sl] = wv * b_xlg[i, sl]
                b_dn[i, :] = jnp.where(pidx == lj, wv, 0.0)
            return carry

        lax.fori_loop(0, csz // 16, group, 0)
        pltpu.sync_copy(b_rows, acc.at[b_dst], add=True)
        pltpu.sync_copy(b_dn, accd.at[b_di], add=True)

    ebase = wid * EPW

    def chunk(kk, carry):
        process(ebase + kk * CH, CH)
        return carry

    lax.fori_loop(0, NFULL, chunk, 0)
    if TAIL:
        process(ebase + NFULL * CH, TAIL)

    plsc.subcore_barrier()
    for kk in range(RPS // CH):
        pltpu.sync_copy(acc.at[pl.ds(base_r + kk * CH, CH)],
                        outn_hbm.at[c, pl.ds(base_r + kk * CH, CH)])
    if RPS % CH:
        pltpu.sync_copy(acc.at[pl.ds(base_r + (RPS // CH) * CH, RPS % CH)],
                        outn_hbm.at[c, pl.ds(base_r + (RPS // CH) * CH,
                                             RPS % CH)])

    @pl.when(s == 0)
    def _():
        pltpu.sync_copy(accd, outd_hbm.at[c])


@functools.lru_cache(maxsize=1)
def _sc_layer_kernel():
    mesh = plsc.VectorSubcoreMesh(core_axis_name="c", subcore_axis_name="s")
    return functools.partial(
        pl.kernel,
        out_type=[
            jax.ShapeDtypeStruct((2, NPAD, H), jnp.float32),
            jax.ShapeDtypeStruct((2, DROWS, 16), jnp.float32),
        ],
        mesh=mesh,
        scratch_types=_sc_scratch(),
    )(_sc_body)


def _sc_layer(xl, xr, src, dst, ea, we, att):
    return _sc_layer_kernel()(xl, xr, src, dst, ea, we, att)


def _sc_scratch():
    return [
        pltpu.VMEM((CH, H), jnp.float32),
        pltpu.VMEM((CH, H), jnp.float32),
        pltpu.VMEM((CH, H), jnp.float32),
        pltpu.VMEM((CH, 16), jnp.float32),
        pltpu.VMEM((CH,), jnp.int32),
        pltpu.VMEM((CH,), jnp.int32),
        pltpu.VMEM((CH,), jnp.int32),
        pltpu.VMEM((CH, 4), jnp.float32),
        pltpu.VMEM((4, H), jnp.float32),
        pltpu.VMEM((H,), jnp.float32),
        pltpu.VMEM_SHARED((NPAD, H), jnp.float32),
        pltpu.VMEM_SHARED((DROWS, 16), jnp.float32),
        pltpu.SemaphoreType.DMA,
    ]


def kernel(x, edge_index, edge_attr, history_actions, effect_history_actions,
           partition, params):
    x = x.astype(jnp.float32)
    ea = edge_attr.astype(jnp.float32)
    src = edge_index[0].astype(jnp.int32)
    dst = edge_index[1].astype(jnp.int32)
    xp = jnp.pad(x, ((0, NPAD - N), (0, 0)))
    ear = jnp.pad(ea.reshape(N, H), ((0, NPAD - N), (0, 0)))
    partp = jnp.pad(partition.astype(jnp.int32), (0, NPAD - (N - 1)),
                    constant_values=-1).reshape(4, 1, BLK)
    hist2 = jnp.concatenate(
        [history_actions.reshape(-1), effect_history_actions], axis=0
    ).astype(jnp.float32).reshape(1, 141)

    c1, c2, c3 = params['conv1'], params['conv2'], params['conv3']

    xl1, xr1, easum = _k0(xp, c1['Wl'], c1['Wr'], ear)
    p1, d1 = _sc_layer(xl1, xr1, src, dst, ea, c1['We'], c1['att'])
    d1 = d1.reshape(2, NPAD)
    xl2, xr2 = _kmid(p1, d1, xl1, xr1, easum, c1['We'],
                     c1['att'].reshape(1, H), c1['b'].reshape(1, H),
                     c2['Wl'], c2['Wr'])
    p2, d2 = _sc_layer(xl2, xr2, src, dst, ea, c2['We'], c2['att'])
    d2 = d2.reshape(2, NPAD)
    xl3, xr3 = _kmid(p2, d2, xl2, xr2, easum, c2['We'],
                     c2['att'].reshape(1, H), c2['b'].reshape(1, H),
                     c3['Wl'], c3['Wr'])
    p3, d3 = _sc_layer(xl3, xr3, src, dst, ea, c3['We'], c3['att'])
    d3 = d3.reshape(2, NPAD)
    sums, cnts = _kfin(p3, d3, xl3, xr3, easum, c3['We'],
                       c3['att'].reshape(1, H), c3['b'].reshape(1, H), partp)
    out = _ktail(sums, cnts, hist2, params['mha'], params['hist'],
                 params['head'])
    return out.reshape(PN, 1)


# speedup vs baseline: 6.0263x; 1.3598x over previous
import functools

import jax
import jax.numpy as jnp
from jax import lax
from jax.experimental import pallas as pl
from jax.experimental.pallas import tpu as pltpu
from jax.experimental.pallas import tpu_sc as plsc

N = 10000
NPAD = 10240
E = 320000
H = 128
PN = 20
ROWW = H
BLK = NPAD // 4
NW = 32
EPW = E // NW
CH = 64
NFULL = EPW // CH
TAIL = EPW - NFULL * CH
RPS = NPAD // 16
DROWS = NPAD // 16

_HI = jax.lax.Precision.HIGHEST


def _dot(a, b):
    return jnp.dot(a, b, precision=_HI)


def _k0_body(x_ref, wl_ref, wr_ref, ea_ref, xl_ref, xr_ref, easum_ref):
    i = pl.program_id(0)
    xb = x_ref[...]
    xl_ref[...] = _dot(xb, wl_ref[...])
    xr_ref[...] = _dot(xb, wr_ref[...])

    @pl.when(i == 0)
    def _():
        easum_ref[...] = jnp.zeros_like(easum_ref)

    easum_ref[...] += jnp.sum(ea_ref[...], axis=0, keepdims=True)


def _k0(xp, wl, wr, ear):
    return pl.pallas_call(
        _k0_body,
        grid=(4,),
        in_specs=[
            pl.BlockSpec((BLK, H), lambda i: (i, 0)),
            pl.BlockSpec((H, H), lambda i: (0, 0)),
            pl.BlockSpec((H, H), lambda i: (0, 0)),
            pl.BlockSpec((BLK, H), lambda i: (i, 0)),
        ],
        out_specs=[
            pl.BlockSpec((BLK, H), lambda i: (i, 0)),
            pl.BlockSpec((BLK, H), lambda i: (i, 0)),
            pl.BlockSpec((1, H), lambda i: (0, 0)),
        ],
        out_shape=[
            jax.ShapeDtypeStruct((NPAD, H), jnp.float32),
            jax.ShapeDtypeStruct((NPAD, H), jnp.float32),
            jax.ShapeDtypeStruct((1, H), jnp.float32),
        ],
    )(xp, wl, wr, ear)


def _layer_out(p_ref, d_ref, xl_ref, xr_ref, easum_ref, we_ref, att_ref,
               b_ref):
    xl = xl_ref[...]
    xr = xr_ref[...]
    sel = (lax.broadcasted_iota(jnp.int32, (H, 4), 0) % 4
           == lax.broadcasted_iota(jnp.int32, (H, 4), 1))
    eam = _dot(easum_ref[...], sel.astype(jnp.float32)) * (1.0 / E)
    eloop = _dot(eam, we_ref[...])
    u = xl + xr + eloop
    m = jnp.maximum(u, 0.2 * u)
    logit = jnp.sum(m * att_ref[...], axis=1, keepdims=True)
    wself = jnp.exp(logit)
    num = p_ref[0] + p_ref[1] + wself * xl
    drow = d_ref[0:1, :] + d_ref[1:2, :]
    dsum = lax.dot_general(drow, jnp.ones((1, 1), jnp.float32),
                           (((0,), (0,)), ((), ())), precision=_HI)
    dn = dsum + wself
    return num / dn + b_ref[...]


def _kmid_body(p_ref, d_ref, xl_ref, xr_ref, easum_ref, we_ref, att_ref,
               b_ref, wl2_ref, wr2_ref, xlo_ref, xro_ref):
    i = pl.program_id(0)
    h = _layer_out(p_ref, d_ref, xl_ref, xr_ref, easum_ref, we_ref, att_ref,
                   b_ref)
    h = jnp.maximum(h, 0.0)
    rowid = lax.broadcasted_iota(jnp.int32, (BLK, H), 0) + i * BLK
    h = jnp.where(rowid < N, h, 0.0)
    xlo_ref[...] = _dot(h, wl2_ref[...])
    xro_ref[...] = _dot(h, wr2_ref[...])


def _kmid(p, d, xl, xr, easum, we, att2, b2, wl2, wr2):
    return pl.pallas_call(
        _kmid_body,
        grid=(4,),
        in_specs=[
            pl.BlockSpec((2, BLK, H), lambda i: (0, i, 0)),
            pl.BlockSpec((2, BLK), lambda i: (0, i)),
            pl.BlockSpec((BLK, H), lambda i: (i, 0)),
            pl.BlockSpec((BLK, H), lambda i: (i, 0)),
            pl.BlockSpec((1, H), lambda i: (0, 0)),
            pl.BlockSpec((4, H), lambda i: (0, 0)),
            pl.BlockSpec((1, H), lambda i: (0, 0)),
            pl.BlockSpec((1, H), lambda i: (0, 0)),
            pl.BlockSpec((H, H), lambda i: (0, 0)),
            pl.BlockSpec((H, H), lambda i: (0, 0)),
        ],
        out_specs=[
            pl.BlockSpec((BLK, H), lambda i: (i, 0)),
            pl.BlockSpec((BLK, H), lambda i: (i, 0)),
        ],
        out_shape=[
            jax.ShapeDtypeStruct((NPAD, H), jnp.float32),
            jax.ShapeDtypeStruct((NPAD, H), jnp.float32),
        ],
    )(p, d, xl, xr, easum, we, att2, b2, wl2, wr2)


def _kfin_body(p_ref, d_ref, xl_ref, xr_ref, easum_ref, we_ref, att_ref,
               b_ref, part_ref, sums_ref, cnts_ref):
    i = pl.program_id(0)
    h = _layer_out(p_ref, d_ref, xl_ref, xr_ref, easum_ref, we_ref, att_ref,
                   b_ref)
    part = part_ref[0]
    pv = lax.broadcasted_iota(jnp.int32, (PN, BLK), 0)
    oh = jnp.where(part == pv, 1.0, 0.0)

    @pl.when(i == 0)
    def _():
        sums_ref[...] = jnp.zeros_like(sums_ref)
        cnts_ref[...] = jnp.zeros_like(cnts_ref)

    sums_ref[...] += _dot(oh, h)
    cnts_ref[...] += jnp.broadcast_to(
        jnp.sum(oh, axis=1, keepdims=True), (PN, H))


def _kfin(p, d, xl, xr, easum, we, att3, b3, partp):
    return pl.pallas_call(
        _kfin_body,
        grid=(4,),
        in_specs=[
            pl.BlockSpec((2, BLK, H), lambda i: (0, i, 0)),
            pl.BlockSpec((2, BLK), lambda i: (0, i)),
            pl.BlockSpec((BLK, H), lambda i: (i, 0)),
            pl.BlockSpec((BLK, H), lambda i: (i, 0)),
            pl.BlockSpec((1, H), lambda i: (0, 0)),
            pl.BlockSpec((4, H), lambda i: (0, 0)),
            pl.BlockSpec((1, H), lambda i: (0, 0)),
            pl.BlockSpec((1, H), lambda i: (0, 0)),
            pl.BlockSpec((1, 1, BLK), lambda i: (i, 0, 0)),
        ],
        out_specs=[
            pl.BlockSpec((PN, H), lambda i: (0, 0)),
            pl.BlockSpec((PN, H), lambda i: (0, 0)),
        ],
        out_shape=[
            jax.ShapeDtypeStruct((PN, H), jnp.float32),
            jax.ShapeDtypeStruct((PN, H), jnp.float32),
        ],
    )(p, d, xl, xr, easum, we, att3, b3, partp)


def _leaky(x):
    return jnp.maximum(x, 0.01 * x)


def _ktail_body(sums_ref, cnts_ref, hist_ref,
                wqt_ref, bq_ref, wkt_ref, bk_ref, wvt_ref, bv_ref,
                wot_ref, bo_ref,
                hw1_ref, hb1_ref, hw2_ref, hb2_ref, hw3_ref, hb3_ref,
                dw1_ref, db1_ref, dw2_ref, db2_ref, dw3_ref, db3_ref,
                out_ref):
    cm = sums_ref[...] / cnts_ref[...]
    q = _dot(cm, wqt_ref[...]) + bq_ref[...]
    k = _dot(cm, wkt_ref[...]) + bk_ref[...]
    v = _dot(cm, wvt_ref[...]) + bv_ref[...]
    hd = H // 4
    heads = []
    for hh in range(4):
        sl = slice(hh * hd, (hh + 1) * hd)
        qh, kh, vh = q[:, sl], k[:, sl], v[:, sl]
        s = lax.dot_general(qh, kh, (((1,), (1,)), ((), ())),
                            precision=_HI) * (1.0 / (hd ** 0.5))
        mx = jnp.max(s, axis=1, keepdims=True)
        ex = jnp.exp(s - mx)
        a = ex / jnp.sum(ex, axis=1, keepdims=True)
        heads.append(_dot(a, vh))
    o128 = jnp.concatenate(heads, axis=1)
    xa = _dot(o128, wot_ref[...]) + bo_ref[...]

    o = _leaky(_dot(hist_ref[...], hw1_ref[...]) + hb1_ref[...])
    o = _leaky(_dot(o, hw2_ref[...]) + hb2_ref[...])
    o = _dot(o, hw3_ref[...]) + hb3_ref[...]

    z = jnp.concatenate([xa, o], axis=0)
    u = db1_ref[...]
    for p_ in range(PN + 1):
        u = u + _dot(z[p_:p_ + 1, :], dw1_ref[p_])
    u = _leaky(u)
    u = _leaky(_dot(u, dw2_ref[...]) + db2_ref[...])
    out_ref[...] = _dot(u, dw3_ref[...]) + db3_ref[...]


def _ktail(sums, cnts, hist2, mm, hp, dp):
    args = [sums, cnts, hist2,
            mm['Wq'].T, mm['bq'].reshape(1, H),
            mm['Wk'].T, mm['bk'].reshape(1, H),
            mm['Wv'].T, mm['bv'].reshape(1, H),
            mm['Wo'].T, mm['bo'].reshape(1, H),
            hp['W1'], hp['b1'].reshape(1, H),
            hp['W2'], hp['b2'].reshape(1, H),
            hp['W3'], hp['b3'].reshape(1, H),
            dp['W1'].reshape(PN + 1, H, H), dp['b1'].reshape(1, H),
            dp['W2'], dp['b2'].reshape(1, H),
            dp['W3'], dp['b3'].reshape(1, PN)]
    return pl.pallas_call(
        _ktail_body,
        in_specs=[pl.BlockSpec(a.shape, functools.partial(lambda nd: (0,) * nd, a.ndim))
                  for a in args],
        out_specs=pl.BlockSpec((1, PN), lambda: (0, 0)),
        out_shape=jax.ShapeDtypeStruct((1, PN), jnp.float32),
    )(*args)


def _sc_body(xl_hbm, xr_hbm, src_hbm, dst_hbm, ea_hbm, we_hbm, att_hbm,
             outn_hbm, outd_hbm,
             xlg0, xrg0, src0, dst0, ea0,
             dnrows, dnidx, wev, attv, acc, accd, sem0):
    c = lax.axis_index("c")
    s = lax.axis_index("s")
    wid = c * 16 + s

    pltpu.sync_copy(we_hbm, wev)
    pltpu.sync_copy(att_hbm, attv)

    zero16 = jnp.zeros((16,), jnp.float32)

    def zrow(r, carry):
        for jj in range(H // 16):
            xlg0[r, pl.ds(jj * 16, 16)] = zero16
        dnrows[r, :] = zero16
        return carry

    lax.fori_loop(0, CH, zrow, 0)
    base_r = s * RPS
    for kk in range(RPS // CH):
        pltpu.sync_copy(xlg0, acc.at[pl.ds(base_r + kk * CH, CH)])
    if RPS % CH:
        pltpu.sync_copy(xlg0.at[pl.ds(0, RPS % CH)],
                        acc.at[pl.ds(base_r + (RPS // CH) * CH, RPS % CH)])

    @pl.when(s == 0)
    def _():
        for kk in range(DROWS // CH):
            pltpu.sync_copy(dnrows, accd.at[pl.ds(kk * CH, CH)])
        if DROWS % CH:
            pltpu.sync_copy(dnrows.at[pl.ds(0, DROWS % CH)],
                            accd.at[pl.ds((DROWS // CH) * CH, DROWS % CH)])

    plsc.subcore_barrier()

    pidx = lax.iota(jnp.int32, 16)
    sets = [(xlg0, xrg0, src0, dst0, ea0, sem0)]

    def load_and_gather(off, csz, st):
        xlg, xrg, srcv, dstv, eav, sem = st
        pltpu.sync_copy(src_hbm.at[pl.ds(off, csz)], srcv.at[pl.ds(0, csz)])
        pltpu.sync_copy(dst_hbm.at[pl.ds(off, csz)], dstv.at[pl.ds(0, csz)])
        pltpu.sync_copy(ea_hbm.at[pl.ds(off, csz)], eav.at[pl.ds(0, csz)])
        h1 = pltpu.async_copy(xl_hbm.at[srcv.at[pl.ds(0, csz)]],
                              xlg.at[pl.ds(0, csz)], sem)
        h2 = pltpu.async_copy(xr_hbm.at[dstv.at[pl.ds(0, csz)]],
                              xrg.at[pl.ds(0, csz)], sem)
        return h1, h2

    def compute_scatter(csz, st):
        xlg, xrg, srcv, dstv, eav, sem = st
        b_dn = dnrows.at[pl.ds(0, csz)]
        b_di = dnidx.at[pl.ds(0, csz)]

        def group(g, carry):
            idxv = dstv[pl.ds(g * 16, 16)]
            dnidx[pl.ds(g * 16, 16)] = lax.shift_right_logical(idxv, 4)
            lv = lax.bitwise_and(idxv, 15)
            for j in range(16):
                i = g * 16 + j
                ev = eav[i, :]
                a0 = ev[0]
                a1 = ev[1]
                a2 = ev[2]
                a3 = ev[3]
                lacc = zero16
                for jj in range(8):
                    sl = pl.ds(jj * 16, 16)
                    u = xlg[i, sl] + xrg[i, sl]
                    u = (u + a0 * wev[0, sl] + a1 * wev[1, sl]
                         + a2 * wev[2, sl] + a3 * wev[3, sl])
                    m = jnp.maximum(u, 0.2 * u)
                    lacc = lacc + m * attv[sl]
                for stp in (1, 2, 4, 8):
                    lacc = lacc + lacc.at[pidx ^ stp].get(
                        mode='promise_in_bounds')
                wv = jnp.exp(lacc)
                lj = lv[j]
                for jj in range(8):
                    sl = pl.ds(jj * 16, 16)
                    xlg[i, sl] = wv * xlg[i, sl]
                dnrows[i, :] = jnp.where(pidx == lj, wv, 0.0)
            return carry

        lax.fori_loop(0, csz // 16, group, 0)
        pltpu.sync_copy(xlg.at[pl.ds(0, csz)], acc.at[dstv.at[pl.ds(0, csz)]],
                        add=True)
        pltpu.sync_copy(b_dn, accd.at[b_di], add=True)

    ebase = wid * EPW

    def chunk(kk, carry):
        h1, h2 = load_and_gather(ebase + kk * CH, CH, sets[0])
        h1.wait()
        h2.wait()
        compute_scatter(CH, sets[0])
        return carry

    lax.fori_loop(0, NFULL, chunk, 0)
    if TAIL:
        h1, h2 = load_and_gather(ebase + NFULL * CH, TAIL, sets[0])
        h1.wait()
        h2.wait()
        compute_scatter(TAIL, sets[0])

    plsc.subcore_barrier()
    for kk in range(RPS // CH):
        pltpu.sync_copy(acc.at[pl.ds(base_r + kk * CH, CH)],
                        outn_hbm.at[c, pl.ds(base_r + kk * CH, CH)])
    if RPS % CH:
        pltpu.sync_copy(acc.at[pl.ds(base_r + (RPS // CH) * CH, RPS % CH)],
                        outn_hbm.at[c, pl.ds(base_r + (RPS // CH) * CH,
                                             RPS % CH)])

    @pl.when(s == 0)
    def _():
        pltpu.sync_copy(accd, outd_hbm.at[c])


@functools.lru_cache(maxsize=1)
def _sc_layer_kernel():
    mesh = plsc.VectorSubcoreMesh(core_axis_name="c", subcore_axis_name="s")
    return functools.partial(
        pl.kernel,
        out_type=[
            jax.ShapeDtypeStruct((2, NPAD, H), jnp.float32),
            jax.ShapeDtypeStruct((2, DROWS, 16), jnp.float32),
        ],
        mesh=mesh,
        scratch_types=_sc_scratch(),
    )(_sc_body)


def _sc_layer(xl, xr, src, dst, ea, we, att):
    return _sc_layer_kernel()(xl, xr, src, dst, ea, we, att)


def _sc_scratch():
    buf_set = [
        pltpu.VMEM((CH, H), jnp.float32),
        pltpu.VMEM((CH, H), jnp.float32),
        pltpu.VMEM((CH,), jnp.int32),
        pltpu.VMEM((CH,), jnp.int32),
        pltpu.VMEM((CH, 4), jnp.float32),
    ]
    return buf_set + [
        pltpu.VMEM((CH, 16), jnp.float32),
        pltpu.VMEM((CH,), jnp.int32),
        pltpu.VMEM((4, H), jnp.float32),
        pltpu.VMEM((H,), jnp.float32),
        pltpu.VMEM_SHARED((NPAD, H), jnp.float32),
        pltpu.VMEM_SHARED((DROWS, 16), jnp.float32),
        pltpu.SemaphoreType.DMA,
    ]


def kernel(x, edge_index, edge_attr, history_actions, effect_history_actions,
           partition, params):
    x = x.astype(jnp.float32)
    ea = edge_attr.astype(jnp.float32)
    src = edge_index[0].astype(jnp.int32)
    dst = edge_index[1].astype(jnp.int32)
    xp = jnp.pad(x, ((0, NPAD - N), (0, 0)))
    ear = jnp.pad(ea.reshape(N, H), ((0, NPAD - N), (0, 0)))
    partp = jnp.pad(partition.astype(jnp.int32), (0, NPAD - (N - 1)),
                    constant_values=-1).reshape(4, 1, BLK)
    hist2 = jnp.concatenate(
        [history_actions.reshape(-1), effect_history_actions], axis=0
    ).astype(jnp.float32).reshape(1, 141)

    c1, c2, c3 = params['conv1'], params['conv2'], params['conv3']

    xl1, xr1, easum = _k0(xp, c1['Wl'], c1['Wr'], ear)
    p1, d1 = _sc_layer(xl1, xr1, src, dst, ea, c1['We'], c1['att'])
    d1 = d1.reshape(2, NPAD)
    xl2, xr2 = _kmid(p1, d1, xl1, xr1, easum, c1['We'],
                     c1['att'].reshape(1, H), c1['b'].reshape(1, H),
                     c2['Wl'], c2['Wr'])
    p2, d2 = _sc_layer(xl2, xr2, src, dst, ea, c2['We'], c2['att'])
    d2 = d2.reshape(2, NPAD)
    xl3, xr3 = _kmid(p2, d2, xl2, xr2, easum, c2['We'],
                     c2['att'].reshape(1, H), c2['b'].reshape(1, H),
                     c3['Wl'], c3['Wr'])
    p3, d3 = _sc_layer(xl3, xr3, src, dst, ea, c3['We'], c3['att'])
    d3 = d3.reshape(2, NPAD)
    sums, cnts = _kfin(p3, d3, xl3, xr3, easum, c3['We'],
                       c3['att'].reshape(1, H), c3['b'].reshape(1, H), partp)
    out = _ktail(sums, cnts, hist2, params['mha'], params['hist'],
                 params['head'])
    return out.reshape(PN, 1)
